# Initial kernel scaffold; baseline (speedup 1.0000x reference)
import jax
import jax.numpy as jnp
from jax.experimental import pallas as pl


def kernel(x, edge_index, edge_attr, epoch, nn1_W1, nn1_b1, nn1_W2, nn1_b2, root1, bias1, nn2_W1, nn2_b1, nn2_W2, nn2_b2, root2, bias2, nn3_W1, nn3_b1, nn3_W2, nn3_b2, root3, bias3):
    raise NotImplementedError("write your pallas kernel here")


# speedup vs baseline: 2.4236x; 2.4236x over previous
import jax
import jax.numpy as jnp
from jax.experimental import pallas as pl
from jax.experimental.pallas import tpu as pltpu

_N = 1024
_E = 2048
_BF = jnp.bfloat16
_F32 = jnp.float32


def _leaky(v):
    return jnp.where(v >= 0, v, 0.01 * v)


def _f32dot(a, b, dims):
    return jax.lax.dot_general(a, b, (dims, ((), ())),
                               preferred_element_type=_F32)


def _prologue_body(ea_ref, w1, b1, w2, b2, w3, b3, h1_ref, h2_ref, h3_ref):
    ea = ea_ref[...].astype(_BF)
    for w_ref, b_ref, out_ref in ((w1, b1, h1_ref), (w2, b2, h2_ref),
                                  (w3, b3, h3_ref)):
        h = _f32dot(ea, w_ref[...].astype(_BF), ((1,), (0,)))
        out_ref[...] = _leaky(h + b_ref[...]).astype(_BF)


def _edge_mlps(edge_attr, W1a, b1a, W1b, b1b, W1c, b1c):
    k1, k2, k3 = W1a.shape[1], W1b.shape[1], W1c.shape[1]
    return pl.pallas_call(
        _prologue_body,
        grid=(1,),
        in_specs=[
            pl.BlockSpec(edge_attr.shape, lambda g: (0, 0)),
            pl.BlockSpec(W1a.shape, lambda g: (0, 0)),
            pl.BlockSpec((1, k1), lambda g: (0, 0)),
            pl.BlockSpec(W1b.shape, lambda g: (0, 0)),
            pl.BlockSpec((1, k2), lambda g: (0, 0)),
            pl.BlockSpec(W1c.shape, lambda g: (0, 0)),
            pl.BlockSpec((1, k3), lambda g: (0, 0)),
        ],
        out_specs=[
            pl.BlockSpec((_E, k1), lambda g: (0, 0)),
            pl.BlockSpec((_E, k2), lambda g: (0, 0)),
            pl.BlockSpec((_E, k3), lambda g: (0, 0)),
        ],
        out_shape=[
            jax.ShapeDtypeStruct((_E, k1), _BF),
            jax.ShapeDtypeStruct((_E, k2), _BF),
            jax.ShapeDtypeStruct((_E, k3), _BF),
        ],
    )(edge_attr, W1a, b1a.reshape(1, k1), W1b, b1b.reshape(1, k2),
      W1c, b1c.reshape(1, k3))


def _make_layer_body(in_, out_, it, n_steps, last):
    def body(xT_ref, src_ref, dst_ref, h_ref, w2_ref, b2_ref, root_ref,
             biasT_ref, *rest):
        if last:
            gT_ref, out_ref, xjT_scr, acc_scr = rest
        else:
            out_ref, xjT_scr, acc_scr = rest
        gi = pl.program_id(0)

        @pl.when(gi == 0)
        def _init():
            ids = jax.lax.broadcasted_iota(jnp.int32, (_E, _N), 1)
            sel = (ids == src_ref[...]).astype(_F32)
            xjT_scr[...] = _f32dot(xT_ref[...], sel, ((1,), (1,)))
            acc_scr[...] = jnp.zeros((out_, _E), _F32)

        wdT = _f32dot(w2_ref[...], h_ref[...], ((0,), (1,)))
        wdT = (wdT + b2_ref[...]).astype(_BF).astype(_F32)
        part = jnp.zeros((out_, _E), _F32)
        for j in range(it):
            xr = xjT_scr[pl.ds(gi * it + j, 1), :].astype(_BF).astype(_F32)
            part = part + xr * wdT[j * out_:(j + 1) * out_, :]
        acc_scr[...] = acc_scr[...] + part

        @pl.when(gi == n_steps - 1)
        def _finish():
            ids = jax.lax.broadcasted_iota(jnp.int32, (_E, _N), 1)
            dsel = (ids == dst_ref[...]).astype(_F32)
            sT = _f32dot(acc_scr[...], dsel, ((1,), (0,)))
            cnt = jnp.sum(dsel, axis=0, keepdims=True)
            mean = sT / jnp.maximum(cnt, 1.0)
            o = mean + _f32dot(root_ref[...].astype(_BF),
                               xT_ref[...].astype(_BF), ((0,), (0,)))
            o = _leaky(o + biasT_ref[...])
            if last:
                z = o + gT_ref[...]
                m = jnp.max(z, axis=0, keepdims=True)
                eq = z == m
                rows = jax.lax.broadcasted_iota(jnp.int32, (out_, _N), 0)
                first = jnp.min(jnp.where(eq, rows, out_), axis=0,
                                keepdims=True)
                out_ref[...] = (rows == first).astype(_F32)
            else:
                out_ref[...] = o

    return body


def _layer(xT, src, dst, h, W2, b2, root, bias, it, gT=None):
    in_ = xT.shape[0]
    K = h.shape[1]
    out_ = root.shape[1]
    n_steps = in_ // it
    w2b = W2.astype(_BF)
    inputs = [xT, src, dst, h, w2b, b2.reshape(in_ * out_, 1), root,
              bias.reshape(out_, 1)]
    in_specs = [
        pl.BlockSpec((in_, _N), lambda g: (0, 0)),
        pl.BlockSpec((_E, 1), lambda g: (0, 0)),
        pl.BlockSpec((_E, 1), lambda g: (0, 0)),
        pl.BlockSpec((_E, K), lambda g: (0, 0)),
        pl.BlockSpec((K, it * out_), lambda g: (0, g)),
        pl.BlockSpec((it * out_, 1), lambda g: (g, 0)),
        pl.BlockSpec((in_, out_), lambda g: (0, 0)),
        pl.BlockSpec((out_, 1), lambda g: (0, 0)),
    ]
    if gT is not None:
        inputs.append(gT)
        in_specs.append(pl.BlockSpec((out_, _N), lambda g: (0, 0)))
    body = _make_layer_body(in_, out_, it, n_steps, last=gT is not None)

    return pl.pallas_call(
        body,
        grid=(n_steps,),
        in_specs=in_specs,
        out_specs=pl.BlockSpec((out_, _N), lambda g: (0, 0)),
        out_shape=jax.ShapeDtypeStruct((out_, _N), _F32),
        scratch_shapes=[
            pltpu.VMEM((in_, _E), _F32),
            pltpu.VMEM((out_, _E), _F32),
        ],
    )(*inputs)


def kernel(x, edge_index, edge_attr, epoch,
           nn1_W1, nn1_b1, nn1_W2, nn1_b2, root1, bias1,
           nn2_W1, nn2_b1, nn2_W2, nn2_b2, root2, bias2,
           nn3_W1, nn3_b1, nn3_W2, nn3_b2, root3, bias3):
    del epoch
    xT = x.T
    src = edge_index[0].reshape(_E, 1)
    dst = edge_index[1].reshape(_E, 1)
    h1, h2, h3 = _edge_mlps(edge_attr, nn1_W1, nn1_b1, nn2_W1, nn2_b1,
                            nn3_W1, nn3_b1)
    d1T = _layer(xT, src, dst, h1, nn1_W2, nn1_b2, root1, bias1, it=2)
    d2T = _layer(d1T, src, dst, h2, nn2_W2, nn2_b2, root2, bias2, it=4)
    gum = jax.random.gumbel(jax.random.key(42), (_N, 64), _F32)
    oneT = _layer(d2T, src, dst, h3, nn3_W2, nn3_b2, root3, bias3, it=16,
                  gT=gum.T)
    return oneT.T
